# Initial kernel scaffold; baseline (speedup 1.0000x reference)
import jax
import jax.numpy as jnp
from jax.experimental import pallas as pl


def kernel(tgt_dec_out, src1_key, src1_map_idx, src2_key, src2_map_idx, out_fc_W, out_fc_b, attn1_W, attn1_b, v1_W, attn2_W, attn2_b, v2_W, lin_W, lin_b):
    raise NotImplementedError("write your pallas kernel here")


# speedup vs baseline: 1.6587x; 1.6587x over previous
import functools

import jax
import jax.numpy as jnp
from jax import lax
from jax.experimental import pallas as pl
from jax.experimental.pallas import tpu as pltpu
from jax.experimental.pallas import tpu_sc as plsc

B, T, S1, S2 = 4, 64, 128, 128
D = 256
V = 32000
EXT = V + S1 + S2
NT = 12
WT = EXT // NT
ROWS = B * T
CS = S1 + S2

_F32 = jnp.float32
_BF16 = jnp.bfloat16


def _dgt(a, b):
    return lax.dot_general(a, b, (((1,), (1,)), ((), ())),
                           preferred_element_type=_F32)


def _ln_last(x, eps=1e-5):
    m = jnp.mean(x, axis=-1, keepdims=True)
    v = jnp.mean((x - m) ** 2, axis=-1, keepdims=True)
    return (x - m) / jnp.sqrt(v + eps)


def _attn_body(xr, k1r, k2r, m1r, m2r, m1cr, m2cr, a1r, b1r, v1r,
               a2r, b2r, v2r, lwr, lbr, zr, p0r):
    x = xr[0]
    tm = jnp.sign(jnp.sum(jnp.abs(x), axis=-1, keepdims=True))
    ones8 = jnp.ones((8, D), dtype=_F32)

    def one_source(kr, ar, br, vr):
        k = kr[0]
        aw = ar[...]
        krs = lax.dot_general(ones8, jnp.abs(k), (((1,), (1,)), ((), ())),
                              preferred_element_type=_F32)
        sm = jnp.sign(krs[0:1, :])
        qp = _dgt(x, aw[:, :D]) + br[...]
        kp = _dgt(k, aw[:, D:])
        e = jnp.tanh(qp[:, None, :] + kp[None, :, :])
        att = jnp.sum(e * vr[...][None, :, :], axis=-1)
        out_att = att * sm * tm
        am = jnp.where(sm == 0.0, -jnp.inf, att)
        mx = jnp.max(am, axis=-1, keepdims=True)
        ex = jnp.exp(am - mx)
        soft = ex / jnp.sum(ex, axis=-1, keepdims=True)
        c = lax.dot_general(soft, k, (((1,), (0,)), ((), ())),
                            preferred_element_type=_F32) * tm
        return _ln_last(out_att), c

    att1n, c1 = one_source(k1r, a1r, b1r, v1r)
    att2n, c2 = one_source(k2r, a2r, b2r, v2r)

    lw = lwr[...]
    l8 = (_dgt(x, lw[:, :D]) + _dgt(c1, lw[:, D:2 * D])
          + _dgt(c2, lw[:, 2 * D:]) + lbr[...])
    l3 = l8[:, 0:3]
    mx = jnp.max(l3, axis=-1, keepdims=True)
    ex = jnp.exp(l3 - mx)
    p = ex / jnp.sum(ex, axis=-1, keepdims=True)

    y = jnp.concatenate([att1n * p[:, 1:2], att2n * p[:, 2:3]], axis=1)
    cols_row = jnp.concatenate([m1r[0], m2r[0]], axis=1)
    cols_col = jnp.concatenate([m1cr[0], m2cr[0]], axis=0)
    eq = (cols_col == cols_row).astype(_F32)
    z = lax.dot_general(y, eq, (((1,), (0,)), ((), ())),
                        preferred_element_type=_F32)
    zr[0] = z
    p0r[...] = p[:, 0:1]


def _attn_call(x, k1, k2, m1, m2, m1c, m2c, a1, b1, v1, a2, b2, v2, lw, lb):
    wspec = lambda shape: pl.BlockSpec(shape, lambda b: (0,) * len(shape))
    return pl.pallas_call(
        _attn_body,
        grid=(B,),
        in_specs=[
            pl.BlockSpec((1, T, D), lambda b: (b, 0, 0)),
            pl.BlockSpec((1, S1, D), lambda b: (b, 0, 0)),
            pl.BlockSpec((1, S2, D), lambda b: (b, 0, 0)),
            pl.BlockSpec((1, 1, S1), lambda b: (b, 0, 0)),
            pl.BlockSpec((1, 1, S2), lambda b: (b, 0, 0)),
            pl.BlockSpec((1, S1, 1), lambda b: (b, 0, 0)),
            pl.BlockSpec((1, S2, 1), lambda b: (b, 0, 0)),
            wspec((D, 2 * D)), wspec((1, D)), wspec((1, D)),
            wspec((D, 2 * D)), wspec((1, D)), wspec((1, D)),
            wspec((8, 3 * D)), wspec((1, 8)),
        ],
        out_specs=[
            pl.BlockSpec((1, T, CS), lambda b: (b, 0, 0)),
            pl.BlockSpec((T, 1), lambda b: (b, 0)),
        ],
        out_shape=[
            jax.ShapeDtypeStruct((B, T, CS), _F32),
            jax.ShapeDtypeStruct((ROWS, 1), _F32),
        ],
    )(x, k1, k2, m1, m2, m1c, m2c, a1, b1, v1, a2, b2, v2, lw, lb)


def _vocab_body(xr, wr, bwtr, bwfr, p0r, outr, mulr, subr):
    c = pl.program_id(0)
    b = pl.program_id(1)

    @pl.when(jnp.logical_and(c == 0, b == 0))
    def _stats():
        wf = wr[...]
        g = lax.dot_general(wf, wf, (((0,), (0,)), ((), ())),
                            preferred_element_type=_F32)
        ones8 = jnp.ones((8, EXT), dtype=_BF16)
        colsum = lax.dot_general(ones8, wf, (((1,), (0,)), ((), ())),
                                 preferred_element_type=_F32)[0:1]
        bwf = bwfr[...]
        bw8 = jnp.broadcast_to(bwf.astype(_BF16), (8, EXT))
        wtb = lax.dot_general(bw8, wf, (((1,), (0,)), ((), ())),
                              preferred_element_type=_F32)[0:1]
        sb = jnp.sum(bwf, axis=1, keepdims=True)
        sb2 = jnp.sum(bwf * bwf, axis=1, keepdims=True)
        xf = xr[...].astype(_F32)
        mu = (jnp.sum(xf * colsum, axis=1, keepdims=True) + sb) / V
        h = lax.dot_general(xf, g, (((1,), (0,)), ((), ())),
                            preferred_element_type=_F32)
        sumsq = (jnp.sum(h * xf, axis=1, keepdims=True)
                 + 2.0 * jnp.sum(xf * wtb, axis=1, keepdims=True)
                 + sb2)
        var = jnp.maximum(sumsq / V - mu * mu, 0.0)
        inv = lax.rsqrt(var + 1e-5)
        p0 = p0r[...]
        mulr[...] = inv * p0
        subr[...] = mu * inv * p0

    xb = xr[pl.ds(b * T, T), :]
    wt = wr[pl.ds(c * WT, WT), :]
    logits = _dgt(xb, wt) + bwtr[...]
    a = mulr[pl.ds(b * T, T), :]
    s = subr[pl.ds(b * T, T), :]
    colid = c * WT + lax.broadcasted_iota(jnp.int32, (T, WT), 1)
    outr[0] = jnp.where(colid < V, logits * a - s, 0.0)


def _vocab_call(xbf, wp, bwp, p0):
    return pl.pallas_call(
        _vocab_body,
        grid=(NT, B),
        in_specs=[
            pl.BlockSpec((ROWS, D), lambda c, b: (0, 0)),
            pl.BlockSpec((EXT, D), lambda c, b: (0, 0)),
            pl.BlockSpec((1, WT), lambda c, b: (0, c)),
            pl.BlockSpec((1, EXT), lambda c, b: (0, 0)),
            pl.BlockSpec((ROWS, 1), lambda c, b: (0, 0)),
        ],
        out_specs=pl.BlockSpec((1, T, WT), lambda c, b: (b, 0, c)),
        out_shape=jax.ShapeDtypeStruct((B, T, EXT), _F32),
        scratch_shapes=[
            pltpu.VMEM((ROWS, 1), _F32),
            pltpu.VMEM((ROWS, 1), _F32),
        ],
    )(xbf, wp, bwp, bwp, p0)


NC, NS = 2, 16
NW = NC * NS
RPW = ROWS // NW


def _sc_scatter_body(outr, zr, colsr, colv, idxv, zv, valv, sem):
    cid = lax.axis_index("c")
    sid = lax.axis_index("s")
    w = sid * NC + cid
    b = w // (NW // B)
    pltpu.sync_copy(colsr.at[b], colv)
    for i in range(RPW):
        r = w * RPW + i
        pltpu.sync_copy(zr.at[r], zv)
        base = r * EXT
        for k in range(2):
            for j in range(8):
                sl = pl.ds(j * 16, 16)
                idxv[k, sl] = colv[k, sl] + base
        for k in range(2):
            pltpu.async_copy(outr.at[idxv.at[k]], valv.at[k], sem).wait()
        for k in range(2):
            for j in range(8):
                sl = pl.ds(j * 16, 16)
                valv[k, sl] = valv[k, sl] + zv[k, sl]
        for k in range(2):
            pltpu.async_copy(valv.at[k], outr.at[idxv.at[k]], sem).wait()


def _sc_scatter(out_ref, z3, cols3):
    mesh = plsc.VectorSubcoreMesh(core_axis_name="c", subcore_axis_name="s",
                                  num_cores=NC, num_subcores=NS)
    f = pl.kernel(
        _sc_scatter_body,
        out_type=(),
        mesh=mesh,
        scratch_types=[
            pltpu.VMEM((2, 128), jnp.int32),
            pltpu.VMEM((2, 128), jnp.int32),
            pltpu.VMEM((2, 128), _F32),
            pltpu.VMEM((2, 128), _F32),
            pltpu.SemaphoreType.DMA,
        ],
    )
    f(out_ref, z3, cols3)


def kernel(tgt_dec_out, src1_key, src1_map_idx, src2_key, src2_map_idx,
           out_fc_W, out_fc_b, attn1_W, attn1_b, v1_W, attn2_W, attn2_b,
           v2_W, lin_W, lin_b):
    x = tgt_dec_out
    m1 = src1_map_idx.reshape(B, 1, S1)
    m2 = src2_map_idx.reshape(B, 1, S2)
    m1c = src1_map_idx.reshape(B, S1, 1)
    m2c = src2_map_idx.reshape(B, S2, 1)
    lw8 = jnp.pad(lin_W, ((0, 5), (0, 0)))
    lb8 = jnp.pad(lin_b, (0, 5)).reshape(1, 8)
    z, p0 = _attn_call(
        x, src1_key, src2_key, m1, m2, m1c, m2c,
        attn1_W, attn1_b.reshape(1, D), v1_W,
        attn2_W, attn2_b.reshape(1, D), v2_W,
        lw8, lb8)

    xbf = x.reshape(ROWS, D).astype(_BF16)
    wp = jnp.pad(out_fc_W, ((0, EXT - V), (0, 0))).astype(_BF16)
    bwp = jnp.pad(out_fc_b, (0, EXT - V)).reshape(1, EXT)
    out = _vocab_call(xbf, wp, bwp, p0)

    cols3 = jnp.concatenate([src1_map_idx, src2_map_idx],
                            axis=1).reshape(B, 2, 128)
    z3 = z.reshape(ROWS, 2, 128)
    ref = jax.new_ref(out.reshape(ROWS * EXT))
    _sc_scatter(ref, z3, cols3)
    return ref[...].reshape(B, T, EXT)
